# baseline (speedup 1.0000x reference)
import jax
import jax.numpy as jnp
from jax.experimental import pallas as pl
from jax.experimental.pallas import tpu as pltpu

_N = 8192
_B = 2048
_NB = _N // _B


def _chamfer_body(gt_ref, gen_ref, out_ref, colmin_ref):
    i = pl.program_id(0)
    gtb = gt_ref[...]
    genb = gen_ref[...]
    sqgt = jnp.sum(gtb * gtb, axis=1, keepdims=True)
    sqgen = jnp.sum(genb * genb, axis=1, keepdims=True)
    sq_hi = sqgen.astype(jnp.bfloat16)
    sq_lo = (sqgen - sq_hi.astype(jnp.float32)).astype(jnp.bfloat16)
    lhs = jnp.concatenate(
        [(gtb + gtb).astype(jnp.bfloat16),
         jnp.full((_B, 2), -1.0, jnp.bfloat16)], axis=1)
    rhs = jnp.concatenate(
        [genb.astype(jnp.bfloat16), sq_hi, sq_lo], axis=1)
    m = jax.lax.dot_general(
        lhs, rhs, (((1,), (1,)), ((), ())),
        preferred_element_type=jnp.float32)
    row_min = sqgt[:, 0] - jnp.max(m, axis=1)
    row_sum = jnp.sum(jnp.maximum(row_min, 0.0)).reshape(1, 1)
    col_min = jnp.min(sqgt - m, axis=0)[None, :]

    @pl.when(i == 0)
    def _init():
        out_ref[...] = row_sum
        colmin_ref[...] = col_min

    @pl.when(i > 0)
    def _acc():
        out_ref[...] += row_sum
        colmin_ref[...] = jnp.minimum(colmin_ref[...], col_min)

    @pl.when(i == _NB - 1)
    def _fin():
        col_sum = jnp.sum(jnp.maximum(colmin_ref[...], 0.0))
        out_ref[...] = (out_ref[...] + col_sum) * (1.0 / _N)


def kernel(gt_points, gen_points):
    out = pl.pallas_call(
        _chamfer_body,
        grid=(_NB,),
        in_specs=[
            pl.BlockSpec((_B, 3), lambda i: (i, 0)),
            pl.BlockSpec((_N, 3), lambda i: (0, 0)),
        ],
        out_specs=pl.BlockSpec((1, 1), lambda i: (0, 0)),
        out_shape=jax.ShapeDtypeStruct((1, 1), jnp.float32),
        scratch_shapes=[pltpu.VMEM((1, _N), jnp.float32)],
        compiler_params=pltpu.CompilerParams(
            dimension_semantics=("arbitrary",),
        ),
    )(gt_points, gen_points)
    return out.reshape(())
